# baseline (speedup 1.0000x reference)
import functools

import jax
import jax.numpy as jnp
from jax import lax
from jax.experimental import pallas as pl
from jax.experimental.pallas import tpu as pltpu
from jax.experimental.pallas import tpu_sc as plsc

LANES = 16


def _build_sc_kernel(B, K, b_per_w, num_cores):
    mesh = plsc.VectorSubcoreMesh(core_axis_name="c", subcore_axis_name="s")

    @functools.partial(
        pl.kernel,
        out_type=jax.ShapeDtypeStruct((B,), jnp.float32),
        mesh=mesh,
        compiler_params=pltpu.CompilerParams(use_tc_tiling_on_sc=False),
        scratch_types=[
            pltpu.VMEM((b_per_w,), jnp.int32),
            pltpu.VMEM((b_per_w,), jnp.int32),
            pltpu.VMEM((b_per_w,), jnp.int32),
            pltpu.VMEM((b_per_w, K), jnp.float32),
            pltpu.VMEM((b_per_w, K), jnp.float32),
            pltpu.VMEM((b_per_w, K), jnp.float32),
            pltpu.VMEM((b_per_w,), jnp.float32),
            pltpu.SemaphoreType.DMA,
            pltpu.SemaphoreType.DMA,
            pltpu.SemaphoreType.DMA,
        ],
    )
    def sc_kernel(idx0_hbm, idx1_hbm, idx2_hbm, f0_hbm, f1_hbm, f2_hbm,
                  out_hbm, idx0_v, idx1_v, idx2_v, r0_v, r1_v, r2_v, out_v,
                  sem0, sem1, sem2):
        wid = lax.axis_index("s") * num_cores + lax.axis_index("c")
        base = wid * b_per_w

        pltpu.sync_copy(idx0_hbm.at[pl.ds(base, b_per_w)], idx0_v)
        pltpu.sync_copy(idx1_hbm.at[pl.ds(base, b_per_w)], idx1_v)
        pltpu.sync_copy(idx2_hbm.at[pl.ds(base, b_per_w)], idx2_v)

        cp0 = pltpu.async_copy(f0_hbm.at[idx0_v], r0_v, sem0)
        cp1 = pltpu.async_copy(f1_hbm.at[idx1_v], r1_v, sem1)
        cp2 = pltpu.async_copy(f2_hbm.at[idx2_v], r2_v, sem2)
        cp0.wait()
        cp1.wait()
        cp2.wait()

        lane = lax.iota(jnp.int32, LANES)
        perms = [jnp.bitwise_xor(lane, s) for s in (8, 4, 2, 1)]

        def body(g, carry):
            vec = jnp.zeros((LANES,), jnp.float32)
            for l in range(LANES):
                b = g * LANES + l
                acc = (r0_v[b, pl.ds(0, LANES)]
                       * r1_v[b, pl.ds(0, LANES)]
                       * r2_v[b, pl.ds(0, LANES)])
                for j in range(1, K // LANES):
                    acc = acc + (r0_v[b, pl.ds(j * LANES, LANES)]
                                 * r1_v[b, pl.ds(j * LANES, LANES)]
                                 * r2_v[b, pl.ds(j * LANES, LANES)])
                for p in perms:
                    acc = acc + jnp.take_along_axis(acc, p, axis=0)
                vec = jnp.where(lane == l, acc, vec)
            out_v[pl.ds(g * LANES, LANES)] = vec
            return carry

        lax.fori_loop(0, b_per_w // LANES, body, 0)

        pltpu.sync_copy(out_v, out_hbm.at[pl.ds(base, b_per_w)])

    return sc_kernel


def kernel(indices, F0, F1, F2):
    B = indices.shape[0]
    K = F0.shape[1]
    info = plsc.get_sparse_core_info()
    num_workers = info.num_cores * info.num_subcores
    b_per_w = B // num_workers
    idx0 = indices[:, 0]
    idx1 = indices[:, 1]
    idx2 = indices[:, 2]
    sc = _build_sc_kernel(B, K, b_per_w, info.num_cores)
    return sc(idx0, idx1, idx2, F0, F1, F2)
